# Initial kernel scaffold; baseline (speedup 1.0000x reference)
import jax
import jax.numpy as jnp
from jax.experimental import pallas as pl


def kernel(enc_conv1_w, enc_conv1_b, enc_conv2_w, enc_conv2_b, enc_fc1_w, enc_fc1_b, enc_heads_w, enc_heads_b, dec_fc_w, dec_fc_b, dec_conv1_w, dec_conv1_b, dec_conv2_w, dec_conv2_b, x, eps):
    raise NotImplementedError("write your pallas kernel here")


# speedup vs baseline: 2.9101x; 2.9101x over previous
import functools

import jax
import jax.numpy as jnp
from jax.experimental import pallas as pl
from jax.experimental.pallas import tpu as pltpu

CDT = jnp.bfloat16


def _mm_kernel(a_ref, b_ref, bias_ref, o_ref, *, act):
    acc = jnp.dot(a_ref[...], b_ref[...], preferred_element_type=jnp.float32)
    acc = acc + bias_ref[...]
    if act == "relu":
        acc = jnp.maximum(acc, 0.0)
    elif act == "sigmoid":
        acc = jax.nn.sigmoid(acc)
    o_ref[...] = acc.astype(o_ref.dtype)


def _mm(a, b, bias, *, tm, tn=None, act=None, out_dtype=CDT):
    M, K = a.shape
    _, N = b.shape
    tn = N if tn is None else tn
    kfn = functools.partial(_mm_kernel, act=act)
    return pl.pallas_call(
        kfn,
        out_shape=jax.ShapeDtypeStruct((M, N), out_dtype),
        grid=(M // tm, N // tn),
        in_specs=[
            pl.BlockSpec((tm, K), lambda i, j: (i, 0)),
            pl.BlockSpec((K, tn), lambda i, j: (0, j)),
            pl.BlockSpec((1, tn), lambda i, j: (0, j)),
        ],
        out_specs=pl.BlockSpec((tm, tn), lambda i, j: (i, j)),
        compiler_params=pltpu.CompilerParams(
            dimension_semantics=("parallel", "parallel")),
    )(a, b, bias.reshape(1, N).astype(jnp.float32))


def _im2col(x, stride, pad):
    B, H, W, C = x.shape
    x = jnp.pad(x, ((0, 0), (pad, pad), (pad, pad), (0, 0)))
    OH = (H + 2 * pad - 3) // stride + 1
    OW = (W + 2 * pad - 3) // stride + 1
    cols = []
    for i in range(3):
        for j in range(3):
            cols.append(x[:, i:i + (OH - 1) * stride + 1:stride,
                          j:j + (OW - 1) * stride + 1:stride, :])
    p = jnp.concatenate(cols, axis=-1)
    return p.reshape(B * OH * OW, 9 * C), OH, OW


def _splitk_kernel(a_ref, b_ref, o_ref):
    @pl.when(pl.program_id(1) == 0)
    def _():
        o_ref[...] = jnp.zeros_like(o_ref)

    o_ref[0] += jnp.dot(a_ref[...], b_ref[...],
                        preferred_element_type=jnp.float32)


def _splitk(a, b, *, tk, nsplit):
    M, K = a.shape
    _, N = b.shape
    nk = K // (nsplit * tk)
    return pl.pallas_call(
        _splitk_kernel,
        out_shape=jax.ShapeDtypeStruct((nsplit, M, N), jnp.float32),
        grid=(nsplit, nk),
        in_specs=[
            pl.BlockSpec((M, tk), lambda s, k: (0, s * nk + k)),
            pl.BlockSpec((tk, N), lambda s, k: (s * nk + k, 0)),
        ],
        out_specs=pl.BlockSpec((1, M, N), lambda s, k: (s, 0, 0)),
        compiler_params=pltpu.CompilerParams(
            dimension_semantics=("parallel", "arbitrary")),
    )(a, b)


def _latent_kernel(p_ref, b1_ref, wh_ref, bh_ref, eps_ref,
                   zm_ref, zlv_ref, z_ref):
    h = p_ref[0]
    for s in range(1, p_ref.shape[0]):
        h = h + p_ref[s]
    h = jnp.maximum(h + b1_ref[...], 0.0)
    zs = jnp.dot(h, wh_ref[...], preferred_element_type=jnp.float32)
    zs = zs + bh_ref[...]
    L = zm_ref.shape[1]
    zm = zs[:, :L]
    zlv = zs[:, L:]
    zm_ref[...] = zm
    zlv_ref[...] = zlv
    z_ref[...] = zm + eps_ref[...] * jnp.exp(0.5 * zlv)


def _latent(partials, b1, wh, bh, eps):
    _, M, N = partials.shape
    H2 = wh.shape[1]
    L = H2 // 2
    return pl.pallas_call(
        _latent_kernel,
        out_shape=(jax.ShapeDtypeStruct((M, L), jnp.float32),
                   jax.ShapeDtypeStruct((M, L), jnp.float32),
                   jax.ShapeDtypeStruct((M, L), jnp.float32)),
    )(partials, b1.reshape(1, N).astype(jnp.float32), wh,
      bh.reshape(1, H2).astype(jnp.float32), eps)


def _dconv_kernel(x_ref, w_ref, b_ref, o_ref, *, OW, act, rows):
    r = pl.program_id(1)
    base = r * rows
    C = x_ref.shape[3]
    Cout = o_ref.shape[2]
    acc = jnp.zeros((rows * OW, Cout), jnp.float32)
    for i in range(3):
        for j in range(3):
            xs = x_ref[0, pl.ds(base + i, rows), pl.ds(j, OW), :]
            xs = xs.reshape(rows * OW, C)
            acc = acc + jnp.dot(xs, w_ref[i * 3 + j],
                                preferred_element_type=jnp.float32)
    acc = acc + b_ref[...]
    if act == "relu":
        acc = jnp.maximum(acc, 0.0)
    else:
        acc = jax.nn.sigmoid(acc)
    o_ref[0] = acc.astype(o_ref.dtype)


def _conv_t(x, wmat, bias, *, pad, out_pad, act, out_dtype, nchunks):
    B, H, W, C = x.shape
    stride = 2
    Cout = wmat.shape[1]
    lo = 2 - pad
    hi = 2 - pad + out_pad
    DH = (H - 1) * stride + 1 + lo + hi
    DW = (W - 1) * stride + 1 + lo + hi
    buf = jnp.zeros((B, DH, DW, C), x.dtype)
    buf = buf.at[:, lo:lo + (H - 1) * stride + 1:stride,
                 lo:lo + (W - 1) * stride + 1:stride, :].set(x)
    OH, OW = DH - 2, DW - 2
    rows = OH // nchunks
    w3 = wmat.reshape(9, C, Cout)
    kfn = functools.partial(_dconv_kernel, OW=OW, act=act, rows=rows)
    out = pl.pallas_call(
        kfn,
        out_shape=jax.ShapeDtypeStruct((B * nchunks, rows * OW, Cout),
                                       out_dtype),
        grid=(B, nchunks),
        in_specs=[
            pl.BlockSpec((1, DH, DW, C), lambda b, r: (b, 0, 0, 0)),
            pl.BlockSpec((9, C, Cout), lambda b, r: (0, 0, 0)),
            pl.BlockSpec((1, Cout), lambda b, r: (0, 0)),
        ],
        out_specs=pl.BlockSpec((1, rows * OW, Cout),
                               lambda b, r: (b * nchunks + r, 0, 0)),
        compiler_params=pltpu.CompilerParams(
            dimension_semantics=("parallel", "arbitrary")),
    )(buf, w3, bias.reshape(1, Cout).astype(jnp.float32))
    return out.reshape(B, OH, OW, Cout)


def kernel(enc_conv1_w, enc_conv1_b, enc_conv2_w, enc_conv2_b,
           enc_fc1_w, enc_fc1_b, enc_heads_w, enc_heads_b,
           dec_fc_w, dec_fc_b, dec_conv1_w, dec_conv1_b,
           dec_conv2_w, dec_conv2_b, x, eps):
    B = x.shape[0]
    xh = x.transpose(0, 2, 3, 1).astype(CDT)

    p1, OH1, OW1 = _im2col(xh, 2, 1)
    h1 = _mm(p1, enc_conv1_w, enc_conv1_b, tm=8000, act="relu")
    h1 = h1.reshape(B, OH1, OW1, -1)
    p2, OH2, OW2 = _im2col(h1, 2, 1)
    h2 = _mm(p2, enc_conv2_w, enc_conv2_b, tm=4000, act="relu")
    flat = h2.reshape(B, -1)

    partials = _splitk(flat, enc_fc1_w, tk=3200, nsplit=2)
    z_mean, z_log_var, z = _latent(partials, enc_fc1_b,
                                   enc_heads_w, enc_heads_b, eps)

    hd = _mm(z.astype(CDT), dec_fc_w, dec_fc_b, tm=B, tn=16000, act="relu")
    hd = hd.reshape(B, 50, 50, 64)
    d1 = _conv_t(hd, dec_conv1_w, dec_conv1_b, pad=1, out_pad=1,
                 act="relu", out_dtype=CDT, nchunks=4)
    d2 = _conv_t(d1, dec_conv2_w, dec_conv2_b, pad=1, out_pad=1,
                 act="sigmoid", out_dtype=jnp.float32, nchunks=8)
    x_rec = d2.transpose(0, 3, 1, 2)
    return x_rec, z_mean, z_log_var
